# initial kernel scaffold (unmeasured)
import jax
import jax.numpy as jnp
from jax import lax
from jax.experimental import pallas as pl
from jax.experimental.pallas import tpu as pltpu

N_DEV = 4
M_PER = 2048
K = 8192
N_PER = 1024
BK = 512
NK = K // BK


def kernel(x, w_mat):
    my_pos = lax.axis_index("i")
    targets = (my_pos + jnp.array([1, 2, 3, 0], dtype=jnp.int32)) % N_DEV

    def body(tgt_ref, x_ref, w_ref, out_ref,
             acc, sbuf, rbuf, stage, send_sems, recv_sems, copy_sems):
        s = pl.program_id(0)
        k = pl.program_id(1)
        my = lax.axis_index("i")
        barrier = pltpu.get_barrier_semaphore()

        @pl.when((s == 0) & (k == 0))
        def _():
            for off in range(1, N_DEV):
                pl.semaphore_signal(
                    barrier, inc=1,
                    device_id=((my + off) % N_DEV,),
                    device_id_type=pl.DeviceIdType.MESH,
                )
            pl.semaphore_wait(barrier, N_DEV - 1)

        partial = jnp.dot(
            x_ref[...].astype(jnp.bfloat16),
            w_ref[...].astype(jnp.bfloat16),
            preferred_element_type=jnp.float32,
        )

        @pl.when(k == 0)
        def _():
            acc[...] = partial

        @pl.when(k > 0)
        def _():
            acc[...] += partial

        @pl.when(k == NK - 1)
        def _():
            @pl.when(s < N_DEV - 1)
            def _():
                sbuf[s] = acc[...].astype(jnp.bfloat16)
                rdma = pltpu.make_async_remote_copy(
                    src_ref=sbuf.at[s],
                    dst_ref=rbuf.at[my],
                    send_sem=send_sems.at[s],
                    recv_sem=recv_sems.at[my],
                    device_id=(tgt_ref[s],),
                    device_id_type=pl.DeviceIdType.MESH,
                )
                rdma.start()

            @pl.when(s == N_DEV - 1)
            def _():
                own_cp = pltpu.make_async_copy(
                    acc,
                    out_ref.at[pl.ds(my * M_PER, M_PER), :],
                    copy_sems.at[N_DEV - 1],
                )
                own_cp.start()

                for j in range(N_DEV - 1):
                    src = (my - 1 - j) % N_DEV
                    recv = pltpu.make_async_remote_copy(
                        src_ref=sbuf.at[0],
                        dst_ref=rbuf.at[src],
                        send_sem=send_sems.at[0],
                        recv_sem=recv_sems.at[src],
                        device_id=(src,),
                        device_id_type=pl.DeviceIdType.MESH,
                    )
                    recv.wait_recv()
                    stage[...] = rbuf[src].astype(jnp.float32)
                    cp = pltpu.make_async_copy(
                        stage,
                        out_ref.at[pl.ds(src * M_PER, M_PER), :],
                        copy_sems.at[j],
                    )
                    cp.start()
                    cp.wait()

                for i in range(N_DEV - 1):
                    snd = pltpu.make_async_remote_copy(
                        src_ref=sbuf.at[i],
                        dst_ref=rbuf.at[my],
                        send_sem=send_sems.at[i],
                        recv_sem=recv_sems.at[my],
                        device_id=((my + 1 + i) % N_DEV,),
                        device_id_type=pl.DeviceIdType.MESH,
                    )
                    snd.wait_send()
                own_cp.wait()

    grid_spec = pltpu.PrefetchScalarGridSpec(
        num_scalar_prefetch=1,
        grid=(N_DEV, NK),
        in_specs=[
            pl.BlockSpec((M_PER, BK), lambda s, k, tgt: (0, k)),
            pl.BlockSpec((BK, N_PER), lambda s, k, tgt: (k, tgt[s])),
        ],
        out_specs=pl.BlockSpec(memory_space=pltpu.ANY),
        scratch_shapes=[
            pltpu.VMEM((M_PER, N_PER), jnp.float32),
            pltpu.VMEM((N_DEV - 1, M_PER, N_PER), jnp.bfloat16),
            pltpu.VMEM((N_DEV, M_PER, N_PER), jnp.bfloat16),
            pltpu.VMEM((M_PER, N_PER), jnp.float32),
            pltpu.SemaphoreType.DMA((N_DEV - 1,)),
            pltpu.SemaphoreType.DMA((N_DEV,)),
            pltpu.SemaphoreType.DMA((N_DEV,)),
        ],
    )

    return pl.pallas_call(
        body,
        grid_spec=grid_spec,
        out_shape=jax.ShapeDtypeStruct((N_DEV * M_PER, N_PER), jnp.float32),
        compiler_params=pltpu.CompilerParams(
            collective_id=0,
            dimension_semantics=("arbitrary", "arbitrary"),
        ),
    )(targets, x, w_mat)


# baseline (device time: 260713 ns/iter reference)
import jax
import jax.numpy as jnp
from jax import lax
from jax.experimental import pallas as pl
from jax.experimental.pallas import tpu as pltpu

N_DEV = 4
M_PER = 2048
K = 8192
N_PER = 1024
BK = 512
NK = K // BK


def kernel(x, w_mat):
    my_pos = lax.axis_index("i")
    targets = (my_pos + jnp.array([1, 2, 3, 0], dtype=jnp.int32)) % N_DEV

    def body(tgt_ref, x_ref, w_ref, out_ref,
             acc, sbuf, rbuf, send_sems, recv_sems, copy_sems):
        s = pl.program_id(0)
        k = pl.program_id(1)
        my = lax.axis_index("i")
        barrier = pltpu.get_barrier_semaphore()

        @pl.when((s == 0) & (k == 0))
        def _():
            for off in range(1, N_DEV):
                pl.semaphore_signal(
                    barrier, inc=1,
                    device_id=((my + off) % N_DEV,),
                    device_id_type=pl.DeviceIdType.MESH,
                )
            pl.semaphore_wait(barrier, N_DEV - 1)

        partial = jnp.dot(
            x_ref[...].astype(jnp.bfloat16),
            w_ref[...].astype(jnp.bfloat16),
            preferred_element_type=jnp.float32,
        )

        @pl.when(k == 0)
        def _():
            acc[...] = partial

        @pl.when(k > 0)
        def _():
            acc[...] += partial

        @pl.when(k == NK - 1)
        def _():
            @pl.when(s < N_DEV - 1)
            def _():
                sbuf[s] = acc[...].astype(jnp.bfloat16)
                rdma = pltpu.make_async_remote_copy(
                    src_ref=sbuf.at[s],
                    dst_ref=rbuf.at[N_DEV - 2 - s],
                    send_sem=send_sems.at[s],
                    recv_sem=recv_sems.at[my],
                    device_id=(tgt_ref[s],),
                    device_id_type=pl.DeviceIdType.MESH,
                )
                rdma.start()

            @pl.when(s == N_DEV - 1)
            def _():
                own_cp = pltpu.make_async_copy(
                    acc,
                    out_ref.at[pl.ds(my * M_PER, M_PER), :],
                    copy_sems.at[N_DEV - 1],
                )
                own_cp.start()
                own_cp.wait()

                for j in range(N_DEV - 1):
                    src = (my - 1 - j) % N_DEV
                    slot = N_DEV - 2 - j
                    recv = pltpu.make_async_remote_copy(
                        src_ref=sbuf.at[0],
                        dst_ref=rbuf.at[slot],
                        send_sem=send_sems.at[0],
                        recv_sem=recv_sems.at[src],
                        device_id=(src,),
                        device_id_type=pl.DeviceIdType.MESH,
                    )
                    recv.wait_recv()
                    acc[...] = rbuf[slot].astype(jnp.float32)
                    cp = pltpu.make_async_copy(
                        acc,
                        out_ref.at[pl.ds(src * M_PER, M_PER), :],
                        copy_sems.at[j],
                    )
                    cp.start()
                    cp.wait()

                for i in range(N_DEV - 1):
                    snd = pltpu.make_async_remote_copy(
                        src_ref=sbuf.at[i],
                        dst_ref=rbuf.at[0],
                        send_sem=send_sems.at[i],
                        recv_sem=recv_sems.at[my],
                        device_id=((my + 1 + i) % N_DEV,),
                        device_id_type=pl.DeviceIdType.MESH,
                    )
                    snd.wait_send()

    grid_spec = pltpu.PrefetchScalarGridSpec(
        num_scalar_prefetch=1,
        grid=(N_DEV, NK),
        in_specs=[
            pl.BlockSpec((M_PER, BK), lambda s, k, tgt: (0, k)),
            pl.BlockSpec((BK, N_PER), lambda s, k, tgt: (k, tgt[s])),
        ],
        out_specs=pl.BlockSpec(memory_space=pl.ANY),
        scratch_shapes=[
            pltpu.VMEM((M_PER, N_PER), jnp.float32),
            pltpu.VMEM((N_DEV - 1, M_PER, N_PER), jnp.bfloat16),
            pltpu.VMEM((N_DEV - 1, M_PER, N_PER), jnp.bfloat16),
            pltpu.SemaphoreType.DMA((N_DEV - 1,)),
            pltpu.SemaphoreType.DMA((N_DEV,)),
            pltpu.SemaphoreType.DMA((N_DEV,)),
        ],
    )

    return pl.pallas_call(
        body,
        grid_spec=grid_spec,
        out_shape=jax.ShapeDtypeStruct((N_DEV * M_PER, N_PER), jnp.float32),
        compiler_params=pltpu.CompilerParams(
            collective_id=0,
            dimension_semantics=("arbitrary", "arbitrary"),
            vmem_limit_bytes=60 * 1024 * 1024,
        ),
    )(targets, x, w_mat)


# device time: 249976 ns/iter; 1.0430x vs baseline; 1.0430x over previous
import jax
import jax.numpy as jnp
from jax import lax
from jax.experimental import pallas as pl
from jax.experimental.pallas import tpu as pltpu

N_DEV = 4
M_PER = 2048
K = 8192
N_PER = 1024
BK = 512
NK = K // BK


def kernel(x, w_mat):
    my_pos = lax.axis_index("i")
    targets = (my_pos + jnp.array([1, 2, 3, 0], dtype=jnp.int32)) % N_DEV

    def body(tgt_ref, x_ref, w_ref, out_ref,
             acc, sbuf, rbuf, send_sems, recv_sems, copy_sems):
        s = pl.program_id(0)
        k = pl.program_id(1)
        my = lax.axis_index("i")
        barrier = pltpu.get_barrier_semaphore()

        @pl.when((s == 0) & (k == 0))
        def _():
            for off in range(1, N_DEV):
                pl.semaphore_signal(
                    barrier, inc=1,
                    device_id=((my + off) % N_DEV,),
                    device_id_type=pl.DeviceIdType.MESH,
                )
            pl.semaphore_wait(barrier, N_DEV - 1)

        partial = jnp.dot(
            x_ref[...].astype(jnp.bfloat16),
            w_ref[...].astype(jnp.bfloat16),
            preferred_element_type=jnp.float32,
        )

        @pl.when(k == 0)
        def _():
            acc[...] = partial

        @pl.when(k > 0)
        def _():
            acc[...] += partial

        PROBE_NO_COMM = True

        @pl.when(k == NK - 1)
        def _():
            @pl.when(s < N_DEV - 1)
            def _():
                sbuf[s] = acc[...].astype(jnp.bfloat16)
                if not PROBE_NO_COMM:
                    rdma = pltpu.make_async_remote_copy(
                        src_ref=sbuf.at[s],
                        dst_ref=rbuf.at[N_DEV - 2 - s],
                        send_sem=send_sems.at[s],
                        recv_sem=recv_sems.at[my],
                        device_id=(tgt_ref[s],),
                        device_id_type=pl.DeviceIdType.MESH,
                    )
                    rdma.start()

            @pl.when(s == N_DEV - 1)
            def _():
                own_cp = pltpu.make_async_copy(
                    acc,
                    out_ref.at[pl.ds(my * M_PER, M_PER), :],
                    copy_sems.at[N_DEV - 1],
                )
                own_cp.start()
                own_cp.wait()

                for j in range(0 if PROBE_NO_COMM else N_DEV - 1):
                    src = (my - 1 - j) % N_DEV
                    slot = N_DEV - 2 - j
                    recv = pltpu.make_async_remote_copy(
                        src_ref=sbuf.at[0],
                        dst_ref=rbuf.at[slot],
                        send_sem=send_sems.at[0],
                        recv_sem=recv_sems.at[src],
                        device_id=(src,),
                        device_id_type=pl.DeviceIdType.MESH,
                    )
                    recv.wait_recv()
                    acc[...] = rbuf[slot].astype(jnp.float32)
                    cp = pltpu.make_async_copy(
                        acc,
                        out_ref.at[pl.ds(src * M_PER, M_PER), :],
                        copy_sems.at[j],
                    )
                    cp.start()
                    cp.wait()

                for i in range(0 if PROBE_NO_COMM else N_DEV - 1):
                    snd = pltpu.make_async_remote_copy(
                        src_ref=sbuf.at[i],
                        dst_ref=rbuf.at[0],
                        send_sem=send_sems.at[i],
                        recv_sem=recv_sems.at[my],
                        device_id=((my + 1 + i) % N_DEV,),
                        device_id_type=pl.DeviceIdType.MESH,
                    )
                    snd.wait_send()

    grid_spec = pltpu.PrefetchScalarGridSpec(
        num_scalar_prefetch=1,
        grid=(N_DEV, NK),
        in_specs=[
            pl.BlockSpec((M_PER, BK), lambda s, k, tgt: (0, k)),
            pl.BlockSpec((BK, N_PER), lambda s, k, tgt: (k, tgt[s])),
        ],
        out_specs=pl.BlockSpec(memory_space=pl.ANY),
        scratch_shapes=[
            pltpu.VMEM((M_PER, N_PER), jnp.float32),
            pltpu.VMEM((N_DEV - 1, M_PER, N_PER), jnp.bfloat16),
            pltpu.VMEM((N_DEV - 1, M_PER, N_PER), jnp.bfloat16),
            pltpu.SemaphoreType.DMA((N_DEV - 1,)),
            pltpu.SemaphoreType.DMA((N_DEV,)),
            pltpu.SemaphoreType.DMA((N_DEV,)),
        ],
    )

    return pl.pallas_call(
        body,
        grid_spec=grid_spec,
        out_shape=jax.ShapeDtypeStruct((N_DEV * M_PER, N_PER), jnp.float32),
        compiler_params=pltpu.CompilerParams(
            collective_id=0,
            dimension_semantics=("arbitrary", "arbitrary"),
            vmem_limit_bytes=60 * 1024 * 1024,
        ),
    )(targets, x, w_mat)


# device time: 245906 ns/iter; 1.0602x vs baseline; 1.0166x over previous
import jax
import jax.numpy as jnp
from jax import lax
from jax.experimental import pallas as pl
from jax.experimental.pallas import tpu as pltpu

N_DEV = 4
M_PER = 2048
K = 8192
N_PER = 1024
BK = 512
NK = K // BK


def kernel(x, w_mat):
    my_pos = lax.axis_index("i")
    targets = (my_pos + jnp.array([1, 2, 3, 0], dtype=jnp.int32)) % N_DEV

    def body(tgt_ref, x_ref, w_ref, out_ref,
             acc, sbuf, rbuf, send_sems, recv_sems, copy_sems):
        s = pl.program_id(0)
        k = pl.program_id(1)
        my = lax.axis_index("i")
        barrier = pltpu.get_barrier_semaphore()

        @pl.when((s == 0) & (k == 0))
        def _():
            for off in range(1, N_DEV):
                pl.semaphore_signal(
                    barrier, inc=1,
                    device_id=((my + off) % N_DEV,),
                    device_id_type=pl.DeviceIdType.MESH,
                )
            pl.semaphore_wait(barrier, N_DEV - 1)

        partial = jnp.dot(
            x_ref[...].astype(jnp.bfloat16),
            w_ref[...].astype(jnp.bfloat16),
            preferred_element_type=jnp.float32,
        )

        @pl.when(k == 0)
        def _():
            acc[...] = partial

        @pl.when(k > 0)
        def _():
            acc[...] += partial

        PROBE_NO_COMM = True

        @pl.when(k == NK - 1)
        def _():
            @pl.when(s < N_DEV - 1)
            def _():
                sbuf[s] = acc[...].astype(jnp.bfloat16)
                if not PROBE_NO_COMM:
                    rdma = pltpu.make_async_remote_copy(
                        src_ref=sbuf.at[s],
                        dst_ref=rbuf.at[N_DEV - 2 - s],
                        send_sem=send_sems.at[s],
                        recv_sem=recv_sems.at[my],
                        device_id=(tgt_ref[s],),
                        device_id_type=pl.DeviceIdType.MESH,
                    )
                    rdma.start()

            @pl.when(s == N_DEV - 1)
            def _():
                own_cp = pltpu.make_async_copy(
                    acc,
                    out_ref.at[pl.ds(my * M_PER, M_PER), :],
                    copy_sems.at[N_DEV - 1],
                )
                own_cp.start()
                own_cp.wait()

                for j in range(0 if PROBE_NO_COMM else N_DEV - 1):
                    src = (my - 1 - j) % N_DEV
                    slot = N_DEV - 2 - j
                    recv = pltpu.make_async_remote_copy(
                        src_ref=sbuf.at[0],
                        dst_ref=rbuf.at[slot],
                        send_sem=send_sems.at[0],
                        recv_sem=recv_sems.at[src],
                        device_id=(src,),
                        device_id_type=pl.DeviceIdType.MESH,
                    )
                    recv.wait_recv()
                    acc[...] = rbuf[slot].astype(jnp.float32)
                    cp = pltpu.make_async_copy(
                        acc,
                        out_ref.at[pl.ds(src * M_PER, M_PER), :],
                        copy_sems.at[j],
                    )
                    cp.start()
                    cp.wait()

                for i in range(0 if PROBE_NO_COMM else N_DEV - 1):
                    snd = pltpu.make_async_remote_copy(
                        src_ref=sbuf.at[i],
                        dst_ref=rbuf.at[0],
                        send_sem=send_sems.at[i],
                        recv_sem=recv_sems.at[my],
                        device_id=((my + 1 + i) % N_DEV,),
                        device_id_type=pl.DeviceIdType.MESH,
                    )
                    snd.wait_send()

    grid_spec = pltpu.PrefetchScalarGridSpec(
        num_scalar_prefetch=1,
        grid=(N_DEV, NK),
        in_specs=[
            pl.BlockSpec((M_PER, BK), lambda s, k, tgt: (0, 0)),
            pl.BlockSpec((BK, N_PER), lambda s, k, tgt: (k, tgt[s])),
        ],
        out_specs=pl.BlockSpec(memory_space=pl.ANY),
        scratch_shapes=[
            pltpu.VMEM((M_PER, N_PER), jnp.float32),
            pltpu.VMEM((N_DEV - 1, M_PER, N_PER), jnp.bfloat16),
            pltpu.VMEM((N_DEV - 1, M_PER, N_PER), jnp.bfloat16),
            pltpu.SemaphoreType.DMA((N_DEV - 1,)),
            pltpu.SemaphoreType.DMA((N_DEV,)),
            pltpu.SemaphoreType.DMA((N_DEV,)),
        ],
    )

    return pl.pallas_call(
        body,
        grid_spec=grid_spec,
        out_shape=jax.ShapeDtypeStruct((N_DEV * M_PER, N_PER), jnp.float32),
        compiler_params=pltpu.CompilerParams(
            collective_id=0,
            dimension_semantics=("arbitrary", "arbitrary"),
            vmem_limit_bytes=60 * 1024 * 1024,
        ),
    )(targets, x, w_mat)


# device time: 226636 ns/iter; 1.1504x vs baseline; 1.0850x over previous
import jax
import jax.numpy as jnp
from jax import lax
from jax.experimental import pallas as pl
from jax.experimental.pallas import tpu as pltpu

N_DEV = 4
M_PER = 2048
K = 8192
N_PER = 1024
BK = 1024
NK = K // BK


def kernel(x, w_mat):
    my_pos = lax.axis_index("i")
    targets = (my_pos + jnp.array([1, 2, 3, 0], dtype=jnp.int32)) % N_DEV

    def body(tgt_ref, x_ref, w_ref, out_ref,
             acc, sbuf, rbuf, send_sems, recv_sems, copy_sems):
        s = pl.program_id(0)
        k = pl.program_id(1)
        my = lax.axis_index("i")
        barrier = pltpu.get_barrier_semaphore()

        @pl.when((s == 0) & (k == 0))
        def _():
            for off in range(1, N_DEV):
                pl.semaphore_signal(
                    barrier, inc=1,
                    device_id=((my + off) % N_DEV,),
                    device_id_type=pl.DeviceIdType.MESH,
                )
            pl.semaphore_wait(barrier, N_DEV - 1)

        partial = jnp.dot(
            x_ref[...].astype(jnp.bfloat16),
            w_ref[...].astype(jnp.bfloat16),
            preferred_element_type=jnp.float32,
        )

        @pl.when(k == 0)
        def _():
            acc[...] = partial

        @pl.when(k > 0)
        def _():
            acc[...] += partial

        @pl.when(k == NK - 1)
        def _():
            @pl.when(s < N_DEV - 1)
            def _():
                slot = s % 2
                @pl.when(s == 2)
                def _():
                    prev = pltpu.make_async_remote_copy(
                        src_ref=sbuf.at[0],
                        dst_ref=rbuf.at[0],
                        send_sem=send_sems.at[0],
                        recv_sem=recv_sems.at[my],
                        device_id=(tgt_ref[0],),
                        device_id_type=pl.DeviceIdType.MESH,
                    )
                    prev.wait_send()

                sbuf[slot] = acc[...].astype(jnp.bfloat16)
                rdma = pltpu.make_async_remote_copy(
                    src_ref=sbuf.at[slot],
                    dst_ref=rbuf.at[N_DEV - 2 - s],
                    send_sem=send_sems.at[s],
                    recv_sem=recv_sems.at[my],
                    device_id=(tgt_ref[s],),
                    device_id_type=pl.DeviceIdType.MESH,
                )
                rdma.start()

            @pl.when(s == N_DEV - 1)
            def _():
                own_cp = pltpu.make_async_copy(
                    acc,
                    out_ref.at[pl.ds(my * M_PER, M_PER), :],
                    copy_sems.at[N_DEV - 1],
                )
                own_cp.start()
                own_cp.wait()

                for j in range(N_DEV - 1):
                    src = (my - 1 - j) % N_DEV
                    slot = N_DEV - 2 - j
                    recv = pltpu.make_async_remote_copy(
                        src_ref=sbuf.at[0],
                        dst_ref=rbuf.at[slot],
                        send_sem=send_sems.at[0],
                        recv_sem=recv_sems.at[src],
                        device_id=(src,),
                        device_id_type=pl.DeviceIdType.MESH,
                    )
                    recv.wait_recv()
                    acc[...] = rbuf[slot].astype(jnp.float32)
                    cp = pltpu.make_async_copy(
                        acc,
                        out_ref.at[pl.ds(src * M_PER, M_PER), :],
                        copy_sems.at[j],
                    )
                    cp.start()
                    cp.wait()

                for i in (1, 2):
                    snd = pltpu.make_async_remote_copy(
                        src_ref=sbuf.at[i % 2],
                        dst_ref=rbuf.at[0],
                        send_sem=send_sems.at[i],
                        recv_sem=recv_sems.at[my],
                        device_id=((my + 1 + i) % N_DEV,),
                        device_id_type=pl.DeviceIdType.MESH,
                    )
                    snd.wait_send()

    grid_spec = pltpu.PrefetchScalarGridSpec(
        num_scalar_prefetch=1,
        grid=(N_DEV, NK),
        in_specs=[
            pl.BlockSpec((M_PER, BK), lambda s, k, tgt: (0, k)),
            pl.BlockSpec((BK, N_PER), lambda s, k, tgt: (k, tgt[s])),
        ],
        out_specs=pl.BlockSpec(memory_space=pl.ANY),
        scratch_shapes=[
            pltpu.VMEM((M_PER, N_PER), jnp.float32),
            pltpu.VMEM((2, M_PER, N_PER), jnp.bfloat16),
            pltpu.VMEM((N_DEV - 1, M_PER, N_PER), jnp.bfloat16),
            pltpu.SemaphoreType.DMA((N_DEV - 1,)),
            pltpu.SemaphoreType.DMA((N_DEV,)),
            pltpu.SemaphoreType.DMA((N_DEV,)),
        ],
    )

    return pl.pallas_call(
        body,
        grid_spec=grid_spec,
        out_shape=jax.ShapeDtypeStruct((N_DEV * M_PER, N_PER), jnp.float32),
        compiler_params=pltpu.CompilerParams(
            collective_id=0,
            dimension_semantics=("arbitrary", "arbitrary"),
            vmem_limit_bytes=63 * 1024 * 1024,
        ),
    )(targets, x, w_mat)
